# baseline (speedup 1.0000x reference)
import functools

import jax
import jax.numpy as jnp
from jax import lax
from jax.experimental import pallas as pl
from jax.experimental.pallas import tpu as pltpu
from jax.experimental.pallas import tpu_sc as plsc

DIM = 256
NUM_CODES = 8192
M_BLK = 1152
N_ROWS = 9216
N_BLOCKS = N_ROWS // M_BLK

_SC_INFO = plsc.get_sparse_core_info()
NC, NS = _SC_INFO.num_cores, _SC_INFO.num_subcores
NW = NC * NS
ROWS_PER_W = N_ROWS // NW
CHUNK = 96
NCHUNK = ROWS_PER_W // CHUNK
IDX_COLS = 96
IDX_ROWS_PER_W = ROWS_PER_W // IDX_COLS


_CHUNKS = ((0, 2736), (2736, 5472), (5472, 8192))


def _rtne_bf16(v):
    i = lax.bitcast_convert_type(v, jnp.uint32)
    r = (i + jnp.uint32(0x7FFF) + ((i >> jnp.uint32(16)) & jnp.uint32(1))) \
        & jnp.uint32(0xFFFF0000)
    return lax.bitcast_convert_type(r, jnp.float32)


def _argmin_body(x_ref, cb_ref, a2_ref, b2_ref, ids_ref):
    x = x_ref[...]
    cb = cb_ref[...]
    s = lax.dot_general(cb, x, (((1,), (1,)), ((), ())),
                        preferred_element_type=jnp.float32)
    d2 = a2_ref[...] - 2.0 * s + b2_ref[...]
    acc = None
    idx = None
    for lo, hi in _CHUNKS:
        blk = d2[lo:hi, :]
        m2 = jnp.min(blk, axis=0, keepdims=True)
        v = jnp.sqrt(jnp.maximum(m2, 0.0))
        ti = lax.bitcast_convert_type(v * v, jnp.int32)
        bound = jnp.full_like(v, -jnp.inf)
        for k in range(-3, 4):
            cand = lax.bitcast_convert_type(ti + jnp.int32(k), jnp.float32)
            ok = jnp.sqrt(jnp.maximum(cand, 0.0)) == v
            bound = jnp.where(ok, jnp.maximum(bound, cand), bound)
        rows = lax.broadcasted_iota(jnp.int32, blk.shape, 0) + jnp.int32(lo)
        i = jnp.min(jnp.where(blk <= bound, rows, jnp.int32(NUM_CODES)),
                    axis=0, keepdims=True)
        if acc is None:
            acc, idx = _rtne_bf16(v), i
        else:
            lt = v < acc
            idx = jnp.where(lt, i, idx)
            acc = jnp.where(lt, _rtne_bf16(v), acc)
    ids_ref[...] = idx.reshape(1, 1, M_BLK)


def _snap_ids(flat, codebook):
    a2 = jnp.sum(flat * flat, axis=1, keepdims=True).reshape(1, N_ROWS)
    b2 = jnp.sum(codebook * codebook, axis=1)[None, :].reshape(NUM_CODES, 1)
    ids3 = pl.pallas_call(
        _argmin_body,
        grid=(N_BLOCKS,),
        in_specs=[
            pl.BlockSpec((M_BLK, DIM), lambda m: (m, 0)),
            pl.BlockSpec((NUM_CODES, DIM), lambda m: (0, 0)),
            pl.BlockSpec((1, M_BLK), lambda m: (0, m)),
            pl.BlockSpec((NUM_CODES, 1), lambda m: (0, 0)),
        ],
        out_specs=pl.BlockSpec((1, 1, M_BLK), lambda m: (m, 0, 0)),
        out_shape=jax.ShapeDtypeStruct((N_BLOCKS, 1, M_BLK), jnp.int32),
    )(flat, codebook, a2, b2)
    return ids3.reshape(N_ROWS)


def _gather_rows(codebook, ids):
    mesh = plsc.VectorSubcoreMesh(core_axis_name="c", subcore_axis_name="s")
    idx3d = ids.reshape(NW, IDX_ROWS_PER_W, IDX_COLS)

    @functools.partial(
        pl.kernel,
        mesh=mesh,
        out_type=jax.ShapeDtypeStruct((N_ROWS, DIM), jnp.float32),
        scratch_types=[
            pltpu.VMEM((IDX_ROWS_PER_W, IDX_COLS), jnp.int32),
            pltpu.VMEM((ROWS_PER_W, DIM), jnp.float32),
            pltpu.SemaphoreType.DMA,
        ],
    )
    def gather(table_hbm, idx_hbm, out_hbm, idx_v, rows_v, sem):
        wid = lax.axis_index("s") * NC + lax.axis_index("c")
        pltpu.sync_copy(idx_hbm.at[wid], idx_v)
        copies = []
        for j in range(NCHUNK):
            copies.append(
                pltpu.async_copy(
                    table_hbm.at[idx_v.at[j]],
                    rows_v.at[pl.ds(j * CHUNK, CHUNK)],
                    sem,
                )
            )
        for c in copies:
            c.wait()
        pltpu.sync_copy(rows_v, out_hbm.at[pl.ds(wid * ROWS_PER_W, ROWS_PER_W)])

    return gather(codebook, idx3d)


def kernel(x, codebook):
    flat = x.reshape(-1, DIM)
    ids = _snap_ids(flat, codebook)
    out = _gather_rows(codebook, ids)
    return out.reshape(x.shape)


# speedup vs baseline: 1.0045x; 1.0045x over previous
import functools

import jax
import jax.numpy as jnp
from jax import lax
from jax.experimental import pallas as pl
from jax.experimental.pallas import tpu as pltpu
from jax.experimental.pallas import tpu_sc as plsc

DIM = 256
NUM_CODES = 8192
M_BLK = 1024
N_ROWS = 9216
N_BLOCKS = N_ROWS // M_BLK

_SC_INFO = plsc.get_sparse_core_info()
NC, NS = _SC_INFO.num_cores, _SC_INFO.num_subcores
NW = NC * NS
ROWS_PER_W = N_ROWS // NW
CHUNK = 96
NCHUNK = ROWS_PER_W // CHUNK
IDX_COLS = 96
IDX_ROWS_PER_W = ROWS_PER_W // IDX_COLS


_CHUNKS = ((0, 2736), (2736, 5472), (5472, 8192))


def _rtne_bf16(v):
    i = lax.bitcast_convert_type(v, jnp.uint32)
    r = (i + jnp.uint32(0x7FFF) + ((i >> jnp.uint32(16)) & jnp.uint32(1))) \
        & jnp.uint32(0xFFFF0000)
    return lax.bitcast_convert_type(r, jnp.float32)


def _argmin_body(x_ref, cb_ref, a2_ref, b2_ref, ids_ref):
    x = x_ref[...]
    cb = cb_ref[...]
    s = lax.dot_general(cb, x, (((1,), (1,)), ((), ())),
                        preferred_element_type=jnp.float32)
    d2 = a2_ref[...] - 2.0 * s + b2_ref[...]
    acc = None
    idx = None
    for lo, hi in _CHUNKS:
        blk = d2[lo:hi, :]
        m2 = jnp.min(blk, axis=0, keepdims=True)
        v = jnp.sqrt(jnp.maximum(m2, 0.0))
        ti = lax.bitcast_convert_type(v * v, jnp.int32)
        bound = jnp.full_like(v, -jnp.inf)
        for k in range(-3, 4):
            cand = lax.bitcast_convert_type(ti + jnp.int32(k), jnp.float32)
            ok = jnp.sqrt(jnp.maximum(cand, 0.0)) == v
            bound = jnp.where(ok, jnp.maximum(bound, cand), bound)
        rows = lax.broadcasted_iota(jnp.int32, blk.shape, 0) + jnp.int32(lo)
        i = jnp.min(jnp.where(blk <= bound, rows, jnp.int32(NUM_CODES)),
                    axis=0, keepdims=True)
        if acc is None:
            acc, idx = _rtne_bf16(v), i
        else:
            lt = v < acc
            idx = jnp.where(lt, i, idx)
            acc = jnp.where(lt, _rtne_bf16(v), acc)
    ids_ref[...] = idx.reshape(1, 1, M_BLK)


def _snap_ids(flat, codebook):
    a2 = jnp.sum(flat * flat, axis=1, keepdims=True).reshape(1, N_ROWS)
    b2 = jnp.sum(codebook * codebook, axis=1)[None, :].reshape(NUM_CODES, 1)
    ids3 = pl.pallas_call(
        _argmin_body,
        grid=(N_BLOCKS,),
        in_specs=[
            pl.BlockSpec((M_BLK, DIM), lambda m: (m, 0)),
            pl.BlockSpec((NUM_CODES, DIM), lambda m: (0, 0)),
            pl.BlockSpec((1, M_BLK), lambda m: (0, m)),
            pl.BlockSpec((NUM_CODES, 1), lambda m: (0, 0)),
        ],
        out_specs=pl.BlockSpec((1, 1, M_BLK), lambda m: (m, 0, 0)),
        out_shape=jax.ShapeDtypeStruct((N_BLOCKS, 1, M_BLK), jnp.int32),
    )(flat, codebook, a2, b2)
    return ids3.reshape(N_ROWS)


def _gather_rows(codebook, ids):
    mesh = plsc.VectorSubcoreMesh(core_axis_name="c", subcore_axis_name="s")
    idx3d = ids.reshape(NW, IDX_ROWS_PER_W, IDX_COLS)

    @functools.partial(
        pl.kernel,
        mesh=mesh,
        out_type=jax.ShapeDtypeStruct((N_ROWS, DIM), jnp.float32),
        scratch_types=[
            pltpu.VMEM((IDX_ROWS_PER_W, IDX_COLS), jnp.int32),
            pltpu.VMEM((ROWS_PER_W, DIM), jnp.float32),
            pltpu.SemaphoreType.DMA,
        ],
    )
    def gather(table_hbm, idx_hbm, out_hbm, idx_v, rows_v, sem):
        wid = lax.axis_index("s") * NC + lax.axis_index("c")
        pltpu.sync_copy(idx_hbm.at[wid], idx_v)
        copies = []
        for j in range(NCHUNK):
            copies.append(
                pltpu.async_copy(
                    table_hbm.at[idx_v.at[j]],
                    rows_v.at[pl.ds(j * CHUNK, CHUNK)],
                    sem,
                )
            )
        for c in copies:
            c.wait()
        pltpu.sync_copy(rows_v, out_hbm.at[pl.ds(wid * ROWS_PER_W, ROWS_PER_W)])

    return gather(codebook, idx3d)


def kernel(x, codebook):
    flat = x.reshape(-1, DIM)
    ids = _snap_ids(flat, codebook)
    out = _gather_rows(codebook, ids)
    return out.reshape(x.shape)
